# baseline (speedup 1.0000x reference)
import functools

import jax
import jax.numpy as jnp
from jax import lax
from jax.experimental import pallas as pl
from jax.experimental.pallas import tpu as pltpu
from jax.experimental.pallas import tpu_sc as plsc

F32 = jnp.float32
BF16 = jnp.bfloat16

E = 600000
N_GRID = 100000
N_MESH = 10000
D = 128
DE = 4

NC = 2
NS = 16
NW = NC * NS

CHUNK = 128
NCHUNK = 147
PER_TILE = CHUNK * NCHUNK
EPAD = NW * PER_TILE

NSLICE = 21
NCS = NCHUNK // NSLICE
ES = EPAD // NSLICE

B_EDGE = 2048
B_NODE = 1000


def _ln(h, g, b):
    mu = jnp.mean(h, axis=-1, keepdims=True)
    var = jnp.mean((h - mu) ** 2, axis=-1, keepdims=True)
    return g * (h - mu) / jnp.sqrt(var + 1e-5) + b


def _dot(a, b):
    return jnp.dot(a, b, preferred_element_type=F32)


def _prep_body(mesh_ref, grid0_ref, wa_ref, wb_ref, mp_ref, gp_ref):
    mp_ref[...] = _dot(mesh_ref[...], wa_ref[...])
    gp_ref[...] = _dot(grid0_ref[...], wb_ref[...])


def _prep(mesh_nfeat, grid0, Wf0a, Wf0b):
    nblk = N_MESH // B_NODE
    return pl.pallas_call(
        _prep_body,
        grid=(nblk,),
        in_specs=[
            pl.BlockSpec((B_NODE, D), lambda i: (i, 0)),
            pl.BlockSpec((B_NODE, D), lambda i: (i, 0)),
            pl.BlockSpec((D, D), lambda i: (0, 0)),
            pl.BlockSpec((D, D), lambda i: (0, 0)),
        ],
        out_specs=[
            pl.BlockSpec((B_NODE, D), lambda i: (i, 0)),
            pl.BlockSpec((B_NODE, D), lambda i: (i, 0)),
        ],
        out_shape=[
            jax.ShapeDtypeStruct((N_MESH, D), F32),
            jax.ShapeDtypeStruct((N_MESH, D), F32),
        ],
    )(mesh_nfeat, grid0, Wf0a, Wf0b)


def _sc_gather_body(src3_hbm, dst3_hbm, mtab_hbm, gtab_hbm,
                    outm_hbm, outg_hbm,
                    idx_s, idx_d, bm0, bm1, bg0, bg1,
                    sm0, sm1, sg0, sg1, swm0, swm1, swg0, swg1):
    cid = lax.axis_index("c")
    sid = lax.axis_index("s")
    wid = sid * NC + cid
    base = wid * (NCS * CHUNK)

    pltpu.sync_copy(src3_hbm.at[wid], idx_s)
    pltpu.sync_copy(dst3_hbm.at[wid], idx_d)

    bm = (bm0, bm1)
    bg = (bg0, bg1)
    sm = (sm0, sm1)
    sg = (sg0, sg1)
    swm = (swm0, swm1)
    swg = (swg0, swg1)
    h = {}

    h["m", 0] = pltpu.async_copy(mtab_hbm.at[idx_s.at[0]], bm[0], sm[0])
    h["g", 0] = pltpu.async_copy(gtab_hbm.at[idx_d.at[0]], bg[0], sg[0])
    for g in range(NCS):
        sl = g % 2
        nsl = (g + 1) % 2
        if g + 1 < NCS:
            if g >= 1:
                h["wm", g - 1].wait()
                h["wg", g - 1].wait()
            h["m", g + 1] = pltpu.async_copy(
                mtab_hbm.at[idx_s.at[g + 1]], bm[nsl], sm[nsl])
            h["g", g + 1] = pltpu.async_copy(
                gtab_hbm.at[idx_d.at[g + 1]], bg[nsl], sg[nsl])
        h["m", g].wait()
        h["g", g].wait()
        off = base + g * CHUNK
        h["wm", g] = pltpu.async_copy(
            bm[sl], outm_hbm.at[pl.ds(off, CHUNK)], swm[sl])
        h["wg", g] = pltpu.async_copy(
            bg[sl], outg_hbm.at[pl.ds(off, CHUNK)], swg[sl])
    h["wm", NCS - 2].wait()
    h["wg", NCS - 2].wait()
    h["wm", NCS - 1].wait()
    h["wg", NCS - 1].wait()


def _sc_gather(src3, dst3, mtab, gtab):
    mesh = plsc.VectorSubcoreMesh(
        core_axis_name="c", subcore_axis_name="s", num_cores=NC,
        num_subcores=NS)
    f = pl.kernel(
        _sc_gather_body,
        out_type=[
            jax.ShapeDtypeStruct((ES, D), F32),
            jax.ShapeDtypeStruct((ES, D), F32),
        ],
        mesh=mesh,
        compiler_params=pltpu.CompilerParams(use_tc_tiling_on_sc=True),
        scratch_types=[
            pltpu.VMEM((NCS, CHUNK), jnp.int32),
            pltpu.VMEM((NCS, CHUNK), jnp.int32),
            pltpu.VMEM((CHUNK, D), F32),
            pltpu.VMEM((CHUNK, D), F32),
            pltpu.VMEM((CHUNK, D), F32),
            pltpu.VMEM((CHUNK, D), F32),
            pltpu.SemaphoreType.DMA,
            pltpu.SemaphoreType.DMA,
            pltpu.SemaphoreType.DMA,
            pltpu.SemaphoreType.DMA,
            pltpu.SemaphoreType.DMA,
            pltpu.SemaphoreType.DMA,
            pltpu.SemaphoreType.DMA,
            pltpu.SemaphoreType.DMA,
        ],
    )
    return f(src3, dst3, mtab, gtab)


def _edge_body(base_rows, eft_ref, gm_ref, gg_ref,
               We0_ref, be0_ref, We1_ref, be1_ref, ge_ref, bge_ref,
               Wf0c_ref, bf0_ref, Wf1_ref, bf1_ref, gf_ref, bgf_ref,
               out_ref):
    i = pl.program_id(0)
    emb = lax.dot_general(eft_ref[...], We0_ref[...],
                          (((0,), (0,)), ((), ())),
                          preferred_element_type=F32)
    u = jax.nn.silu(emb + be0_ref[...])
    h = _dot(u, We1_ref[...]) + be1_ref[...]
    efeat = _ln(h, ge_ref[...], bge_ref[...])
    pre = (_dot(efeat, Wf0c_ref[...]) + bf0_ref[...]
           + gm_ref[...] + gg_ref[...])
    h2 = _dot(jax.nn.silu(pre), Wf1_ref[...]) + bf1_ref[...]
    e_upd = _ln(h2, gf_ref[...], bgf_ref[...])
    rows = (base_rows + i * B_EDGE
            + lax.broadcasted_iota(jnp.int32, (B_EDGE, 1), 0))
    out_ref[...] = jnp.where(rows < E, e_upd, 0.0)


def _edge(s, eft_p, gath_m, gath_g, We0, be0, We1, be1, ge, bge, Wf0c, bf0,
          Wf1, bf1, gf, bgf):
    nblk = ES // B_EDGE
    base_blk = s * nblk
    full = lambda shape: pl.BlockSpec(shape, lambda i: (0, 0))
    return pl.pallas_call(
        functools.partial(_edge_body, s * ES),
        grid=(nblk,),
        in_specs=[
            pl.BlockSpec((DE, B_EDGE), lambda i: (0, base_blk + i)),
            pl.BlockSpec((B_EDGE, D), lambda i: (i, 0)),
            pl.BlockSpec((B_EDGE, D), lambda i: (i, 0)),
            full((DE, D)), full((1, D)), full((D, D)), full((1, D)),
            full((1, D)), full((1, D)),
            full((D, D)), full((1, D)), full((D, D)), full((1, D)),
            full((1, D)), full((1, D)),
        ],
        out_specs=pl.BlockSpec((B_EDGE, D), lambda i: (i, 0)),
        out_shape=jax.ShapeDtypeStruct((ES, D), F32),
    )(eft_p, gath_m, gath_g, We0, be0, We1, be1, ge, bge, Wf0c, bf0,
      Wf1, bf1, gf, bgf)


def _sc_scatter_body(eupd_hbm, dst3_hbm, zeros_hbm, agg_hbm,
                     idx_t, be0, be1, agg_s, se0, se1):
    cid = lax.axis_index("c")
    sid = lax.axis_index("s")
    wid = sid * NC + cid
    base = wid * (NCS * CHUNK)

    pltpu.sync_copy(dst3_hbm.at[wid], idx_t)

    @pl.when(sid == 0)
    def _():
        pltpu.sync_copy(zeros_hbm, agg_s)

    plsc.subcore_barrier()

    be = (be0, be1)
    se = (se0, se1)
    h = {}
    h[0] = pltpu.async_copy(eupd_hbm.at[pl.ds(base, CHUNK)], be[0], se[0])
    for g in range(NCS):
        sl = g % 2
        nsl = (g + 1) % 2
        if g + 1 < NCS:
            h[g + 1] = pltpu.async_copy(
                eupd_hbm.at[pl.ds(base + (g + 1) * CHUNK, CHUNK)],
                be[nsl], se[nsl])
        h[g].wait()
        pltpu.sync_copy(be[sl], agg_s.at[idx_t.at[g]], add=True)

    plsc.subcore_barrier()

    @pl.when(sid == 0)
    def _():
        pltpu.sync_copy(agg_s, agg_hbm.at[cid])


def _sc_scatter(e_upd, dst3, zeros):
    mesh = plsc.VectorSubcoreMesh(
        core_axis_name="c", subcore_axis_name="s", num_cores=NC,
        num_subcores=NS)
    f = pl.kernel(
        _sc_scatter_body,
        out_type=jax.ShapeDtypeStruct((NC, N_MESH, D), F32),
        mesh=mesh,
        compiler_params=pltpu.CompilerParams(use_tc_tiling_on_sc=True),
        scratch_types=[
            pltpu.VMEM((NCS, CHUNK), jnp.int32),
            pltpu.VMEM((CHUNK, D), F32),
            pltpu.VMEM((CHUNK, D), F32),
            pltpu.VMEM_SHARED((N_MESH, D), F32),
            pltpu.SemaphoreType.DMA,
            pltpu.SemaphoreType.DMA,
        ],
    )
    return f(e_upd, dst3, zeros)


NAGG = NSLICE * NC


def _node_plain_body(grid_ref, Wn0a_ref, bn0_ref, Wn1_ref, bn1_ref,
                     gn_ref, bgn_ref, out_ref):
    g = grid_ref[...]
    pre = _dot(g, Wn0a_ref[...]) + bn0_ref[...]
    h = _dot(jax.nn.silu(pre), Wn1_ref[...]) + bn1_ref[...]
    out_ref[...] = g + _ln(h, gn_ref[...], bgn_ref[...])


def _node_plain(grid_nfeat, Wn0a, bn0, Wn1, bn1, gn, bgn):
    nblk = N_GRID // B_NODE
    full = lambda shape: pl.BlockSpec(shape, lambda i: (0, 0))
    return pl.pallas_call(
        _node_plain_body,
        grid=(nblk,),
        in_specs=[
            pl.BlockSpec((B_NODE, D), lambda i: (i, 0)),
            full((D, D)), full((1, D)), full((D, D)),
            full((1, D)), full((1, D)), full((1, D)),
        ],
        out_specs=pl.BlockSpec((B_NODE, D), lambda i: (i, 0)),
        out_shape=jax.ShapeDtypeStruct((N_GRID, D), F32),
    )(grid_nfeat, Wn0a, bn0, Wn1, bn1, gn, bgn)


def _node_agg_body(grid_ref, *rest):
    agg_refs = rest[:NAGG]
    (Wn0a_ref, Wn0b_ref, bn0_ref, Wn1_ref, bn1_ref,
     gn_ref, bgn_ref, out_ref) = rest[NAGG:]
    g = grid_ref[...]
    agg = agg_refs[0][...]
    for a in agg_refs[1:]:
        agg = agg + a[...]
    pre = _dot(g, Wn0a_ref[...]) + _dot(agg, Wn0b_ref[...]) + bn0_ref[...]
    h = _dot(jax.nn.silu(pre), Wn1_ref[...]) + bn1_ref[...]
    out_ref[...] = g + _ln(h, gn_ref[...], bgn_ref[...])


def _node_agg(grid0, aggs, Wn0a, Wn0b, bn0, Wn1, bn1, gn, bgn):
    nblk = N_MESH // B_NODE
    full = lambda shape: pl.BlockSpec(shape, lambda i: (0, 0))
    blk = pl.BlockSpec((B_NODE, D), lambda i: (i, 0))
    return pl.pallas_call(
        _node_agg_body,
        grid=(nblk,),
        in_specs=[
            blk,
            *([blk] * NAGG),
            full((D, D)), full((D, D)), full((1, D)), full((D, D)),
            full((1, D)), full((1, D)), full((1, D)),
        ],
        out_specs=blk,
        out_shape=jax.ShapeDtypeStruct((N_MESH, D), F32),
    )(grid0, *aggs, Wn0a, Wn0b, bn0, Wn1, bn1, gn, bgn)


def kernel(grid_nfeat, mesh_nfeat, edge_index, mesh2grid_efeat,
           We0, be0, We1, be1, ge, bge,
           Wf0, bf0, Wf1, bf1, gf, bgf,
           Wn0, bn0, Wn1, bn1, gn, bgn):
    src = edge_index[0].astype(jnp.int32)
    dst = edge_index[1].astype(jnp.int32)
    src4 = jnp.pad(src, (0, EPAD - E)).reshape(NSLICE, NW, NCS, CHUNK)
    dst4 = jnp.pad(dst, (0, EPAD - E)).reshape(NSLICE, NW, NCS, CHUNK)
    eft_p = jnp.pad(mesh2grid_efeat.T, ((0, 0), (0, EPAD - E)))

    Wf0a, Wf0b, Wf0c = Wf0[:D], Wf0[D:2 * D], Wf0[2 * D:]
    Wn0a, Wn0b = Wn0[:D], Wn0[D:]
    r = lambda v: v.reshape(1, D)
    grid0 = grid_nfeat[:N_MESH]

    mtab, gtab = _prep(mesh_nfeat, grid0, Wf0a, Wf0b)
    zeros = jnp.zeros((N_MESH, D), F32)
    aggs = []
    for s in range(NSLICE):
        gath_m, gath_g = _sc_gather(src4[s], dst4[s], mtab, gtab)
        e_upd = _edge(s, eft_p, gath_m, gath_g, We0, r(be0), We1, r(be1),
                      r(ge), r(bge), Wf0c, r(bf0), Wf1, r(bf1),
                      r(gf), r(bgf))
        aggp = _sc_scatter(e_upd, dst4[s], zeros)
        aggs.extend([aggp[0], aggp[1]])
    out_plain = _node_plain(grid_nfeat, Wn0a, r(bn0), Wn1, r(bn1),
                            r(gn), r(bgn))
    out_agg = _node_agg(grid0, aggs, Wn0a, Wn0b, r(bn0),
                        Wn1, r(bn1), r(gn), r(bgn))
    return lax.dynamic_update_slice(out_plain, out_agg, (0, 0))


# speedup vs baseline: 1.1973x; 1.1973x over previous
import functools

import jax
import jax.numpy as jnp
from jax import lax
from jax.experimental import pallas as pl
from jax.experimental.pallas import tpu as pltpu
from jax.experimental.pallas import tpu_sc as plsc

F32 = jnp.float32
BF16 = jnp.bfloat16

E = 600000
N_GRID = 100000
N_MESH = 10000
D = 128
DE = 4

NC = 2
NS = 16
NW = NC * NS

CHUNK = 128
NCHUNK = 147
PER_TILE = CHUNK * NCHUNK
EPAD = NW * PER_TILE

NSLICE = 3
NCS = NCHUNK // NSLICE
ES = EPAD // NSLICE

B_EDGE = 2048
B_NODE = 1000


def _ln(h, g, b):
    mu = jnp.mean(h, axis=-1, keepdims=True)
    var = jnp.mean((h - mu) ** 2, axis=-1, keepdims=True)
    return g * (h - mu) / jnp.sqrt(var + 1e-5) + b


def _dot(a, b):
    return jnp.dot(a, b, preferred_element_type=F32)


def _prep_body(mesh_ref, grid0_ref, wa_ref, wb_ref, mp_ref, gp_ref):
    mp_ref[...] = _dot(mesh_ref[...], wa_ref[...])
    gp_ref[...] = _dot(grid0_ref[...], wb_ref[...])


def _prep(mesh_nfeat, grid0, Wf0a, Wf0b):
    nblk = N_MESH // B_NODE
    return pl.pallas_call(
        _prep_body,
        grid=(nblk,),
        in_specs=[
            pl.BlockSpec((B_NODE, D), lambda i: (i, 0)),
            pl.BlockSpec((B_NODE, D), lambda i: (i, 0)),
            pl.BlockSpec((D, D), lambda i: (0, 0)),
            pl.BlockSpec((D, D), lambda i: (0, 0)),
        ],
        out_specs=[
            pl.BlockSpec((B_NODE, D), lambda i: (i, 0)),
            pl.BlockSpec((B_NODE, D), lambda i: (i, 0)),
        ],
        out_shape=[
            jax.ShapeDtypeStruct((N_MESH, D), F32),
            jax.ShapeDtypeStruct((N_MESH, D), F32),
        ],
    )(mesh_nfeat, grid0, Wf0a, Wf0b)


def _sc_gather_body(src3_hbm, dst3_hbm, mtab_hbm, gtab_hbm,
                    outm_hbm, outg_hbm,
                    idx_s, idx_d, bm0, bm1, bg0, bg1,
                    sm0, sm1, sg0, sg1, swm0, swm1, swg0, swg1):
    cid = lax.axis_index("c")
    sid = lax.axis_index("s")
    wid = sid * NC + cid
    base = wid * (NCS * CHUNK)

    pltpu.sync_copy(src3_hbm.at[wid], idx_s)
    pltpu.sync_copy(dst3_hbm.at[wid], idx_d)

    bm = (bm0, bm1)
    bg = (bg0, bg1)
    sm = (sm0, sm1)
    sg = (sg0, sg1)
    swm = (swm0, swm1)
    swg = (swg0, swg1)
    h = {}

    h["m", 0] = pltpu.async_copy(mtab_hbm.at[idx_s.at[0]], bm[0], sm[0])
    h["g", 0] = pltpu.async_copy(gtab_hbm.at[idx_d.at[0]], bg[0], sg[0])
    for g in range(NCS):
        sl = g % 2
        nsl = (g + 1) % 2
        if g + 1 < NCS:
            if g >= 1:
                h["wm", g - 1].wait()
                h["wg", g - 1].wait()
            h["m", g + 1] = pltpu.async_copy(
                mtab_hbm.at[idx_s.at[g + 1]], bm[nsl], sm[nsl])
            h["g", g + 1] = pltpu.async_copy(
                gtab_hbm.at[idx_d.at[g + 1]], bg[nsl], sg[nsl])
        h["m", g].wait()
        h["g", g].wait()
        off = base + g * CHUNK
        h["wm", g] = pltpu.async_copy(
            bm[sl], outm_hbm.at[pl.ds(off, CHUNK)], swm[sl])
        h["wg", g] = pltpu.async_copy(
            bg[sl], outg_hbm.at[pl.ds(off, CHUNK)], swg[sl])
    h["wm", NCS - 2].wait()
    h["wg", NCS - 2].wait()
    h["wm", NCS - 1].wait()
    h["wg", NCS - 1].wait()


def _sc_gather(src3, dst3, mtab, gtab):
    mesh = plsc.VectorSubcoreMesh(
        core_axis_name="c", subcore_axis_name="s", num_cores=NC,
        num_subcores=NS)
    f = pl.kernel(
        _sc_gather_body,
        out_type=[
            jax.ShapeDtypeStruct((ES, D), F32),
            jax.ShapeDtypeStruct((ES, D), F32),
        ],
        mesh=mesh,
        compiler_params=pltpu.CompilerParams(use_tc_tiling_on_sc=True),
        scratch_types=[
            pltpu.VMEM((NCS, CHUNK), jnp.int32),
            pltpu.VMEM((NCS, CHUNK), jnp.int32),
            pltpu.VMEM((CHUNK, D), F32),
            pltpu.VMEM((CHUNK, D), F32),
            pltpu.VMEM((CHUNK, D), F32),
            pltpu.VMEM((CHUNK, D), F32),
            pltpu.SemaphoreType.DMA,
            pltpu.SemaphoreType.DMA,
            pltpu.SemaphoreType.DMA,
            pltpu.SemaphoreType.DMA,
            pltpu.SemaphoreType.DMA,
            pltpu.SemaphoreType.DMA,
            pltpu.SemaphoreType.DMA,
            pltpu.SemaphoreType.DMA,
        ],
    )
    return f(src3, dst3, mtab, gtab)


def _edge_body(base_rows, eft_ref, gm_ref, gg_ref,
               We0_ref, be0_ref, We1_ref, be1_ref, ge_ref, bge_ref,
               Wf0c_ref, bf0_ref, Wf1_ref, bf1_ref, gf_ref, bgf_ref,
               out_ref):
    i = pl.program_id(0)
    emb = lax.dot_general(eft_ref[...], We0_ref[...],
                          (((0,), (0,)), ((), ())),
                          preferred_element_type=F32)
    u = jax.nn.silu(emb + be0_ref[...])
    h = _dot(u, We1_ref[...]) + be1_ref[...]
    efeat = _ln(h, ge_ref[...], bge_ref[...])
    pre = (_dot(efeat, Wf0c_ref[...]) + bf0_ref[...]
           + gm_ref[...] + gg_ref[...])
    h2 = _dot(jax.nn.silu(pre), Wf1_ref[...]) + bf1_ref[...]
    e_upd = _ln(h2, gf_ref[...], bgf_ref[...])
    rows = (base_rows + i * B_EDGE
            + lax.broadcasted_iota(jnp.int32, (B_EDGE, 1), 0))
    out_ref[...] = jnp.where(rows < E, e_upd, 0.0)


def _edge(s, eft_p, gath_m, gath_g, We0, be0, We1, be1, ge, bge, Wf0c, bf0,
          Wf1, bf1, gf, bgf):
    nblk = ES // B_EDGE
    base_blk = s * nblk
    full = lambda shape: pl.BlockSpec(shape, lambda i: (0, 0))
    return pl.pallas_call(
        functools.partial(_edge_body, s * ES),
        grid=(nblk,),
        in_specs=[
            pl.BlockSpec((DE, B_EDGE), lambda i: (0, base_blk + i)),
            pl.BlockSpec((B_EDGE, D), lambda i: (i, 0)),
            pl.BlockSpec((B_EDGE, D), lambda i: (i, 0)),
            full((DE, D)), full((1, D)), full((D, D)), full((1, D)),
            full((1, D)), full((1, D)),
            full((D, D)), full((1, D)), full((D, D)), full((1, D)),
            full((1, D)), full((1, D)),
        ],
        out_specs=pl.BlockSpec((B_EDGE, D), lambda i: (i, 0)),
        out_shape=jax.ShapeDtypeStruct((ES, D), F32),
    )(eft_p, gath_m, gath_g, We0, be0, We1, be1, ge, bge, Wf0c, bf0,
      Wf1, bf1, gf, bgf)


def _sc_scatter_body(eupd_hbm, dst3_hbm, zeros_hbm, agg_hbm,
                     idx_t, be0, be1, agg_s, se0, se1):
    cid = lax.axis_index("c")
    sid = lax.axis_index("s")
    wid = sid * NC + cid
    base = wid * (NCS * CHUNK)

    pltpu.sync_copy(dst3_hbm.at[wid], idx_t)

    @pl.when(sid == 0)
    def _():
        pltpu.sync_copy(zeros_hbm, agg_s)

    plsc.subcore_barrier()

    be = (be0, be1)
    se = (se0, se1)
    h = {}
    h[0] = pltpu.async_copy(eupd_hbm.at[pl.ds(base, CHUNK)], be[0], se[0])
    for g in range(NCS):
        sl = g % 2
        nsl = (g + 1) % 2
        if g + 1 < NCS:
            h[g + 1] = pltpu.async_copy(
                eupd_hbm.at[pl.ds(base + (g + 1) * CHUNK, CHUNK)],
                be[nsl], se[nsl])
        h[g].wait()
        pltpu.sync_copy(be[sl], agg_s.at[idx_t.at[g]], add=True)

    plsc.subcore_barrier()

    @pl.when(sid == 0)
    def _():
        pltpu.sync_copy(agg_s, agg_hbm.at[cid])


def _sc_scatter(e_upd, dst3, zeros):
    mesh = plsc.VectorSubcoreMesh(
        core_axis_name="c", subcore_axis_name="s", num_cores=NC,
        num_subcores=NS)
    f = pl.kernel(
        _sc_scatter_body,
        out_type=jax.ShapeDtypeStruct((NC, N_MESH, D), F32),
        mesh=mesh,
        compiler_params=pltpu.CompilerParams(use_tc_tiling_on_sc=True),
        scratch_types=[
            pltpu.VMEM((NCS, CHUNK), jnp.int32),
            pltpu.VMEM((CHUNK, D), F32),
            pltpu.VMEM((CHUNK, D), F32),
            pltpu.VMEM_SHARED((N_MESH, D), F32),
            pltpu.SemaphoreType.DMA,
            pltpu.SemaphoreType.DMA,
        ],
    )
    return f(e_upd, dst3, zeros)


NAGG = NSLICE * NC


def _node_plain_body(grid_ref, Wn0a_ref, bn0_ref, Wn1_ref, bn1_ref,
                     gn_ref, bgn_ref, out_ref):
    g = grid_ref[...]
    pre = _dot(g, Wn0a_ref[...]) + bn0_ref[...]
    h = _dot(jax.nn.silu(pre), Wn1_ref[...]) + bn1_ref[...]
    out_ref[...] = g + _ln(h, gn_ref[...], bgn_ref[...])


def _node_plain(grid_nfeat, Wn0a, bn0, Wn1, bn1, gn, bgn):
    nblk = N_GRID // B_NODE
    full = lambda shape: pl.BlockSpec(shape, lambda i: (0, 0))
    return pl.pallas_call(
        _node_plain_body,
        grid=(nblk,),
        in_specs=[
            pl.BlockSpec((B_NODE, D), lambda i: (i, 0)),
            full((D, D)), full((1, D)), full((D, D)),
            full((1, D)), full((1, D)), full((1, D)),
        ],
        out_specs=pl.BlockSpec((B_NODE, D), lambda i: (i, 0)),
        out_shape=jax.ShapeDtypeStruct((N_GRID, D), F32),
    )(grid_nfeat, Wn0a, bn0, Wn1, bn1, gn, bgn)


def _node_agg_body(grid_ref, *rest):
    agg_refs = rest[:NAGG]
    (Wn0a_ref, Wn0b_ref, bn0_ref, Wn1_ref, bn1_ref,
     gn_ref, bgn_ref, out_ref) = rest[NAGG:]
    g = grid_ref[...]
    agg = agg_refs[0][...]
    for a in agg_refs[1:]:
        agg = agg + a[...]
    pre = _dot(g, Wn0a_ref[...]) + _dot(agg, Wn0b_ref[...]) + bn0_ref[...]
    h = _dot(jax.nn.silu(pre), Wn1_ref[...]) + bn1_ref[...]
    out_ref[...] = g + _ln(h, gn_ref[...], bgn_ref[...])


def _node_agg(grid0, aggs, Wn0a, Wn0b, bn0, Wn1, bn1, gn, bgn):
    nblk = N_MESH // B_NODE
    full = lambda shape: pl.BlockSpec(shape, lambda i: (0, 0))
    blk = pl.BlockSpec((B_NODE, D), lambda i: (i, 0))
    return pl.pallas_call(
        _node_agg_body,
        grid=(nblk,),
        in_specs=[
            blk,
            *([blk] * NAGG),
            full((D, D)), full((D, D)), full((1, D)), full((D, D)),
            full((1, D)), full((1, D)), full((1, D)),
        ],
        out_specs=blk,
        out_shape=jax.ShapeDtypeStruct((N_MESH, D), F32),
    )(grid0, *aggs, Wn0a, Wn0b, bn0, Wn1, bn1, gn, bgn)


def kernel(grid_nfeat, mesh_nfeat, edge_index, mesh2grid_efeat,
           We0, be0, We1, be1, ge, bge,
           Wf0, bf0, Wf1, bf1, gf, bgf,
           Wn0, bn0, Wn1, bn1, gn, bgn):
    src = edge_index[0].astype(jnp.int32)
    dst = edge_index[1].astype(jnp.int32)
    src4 = jnp.pad(src, (0, EPAD - E)).reshape(NSLICE, NW, NCS, CHUNK)
    dst4 = jnp.pad(dst, (0, EPAD - E)).reshape(NSLICE, NW, NCS, CHUNK)
    eft_p = jnp.pad(mesh2grid_efeat.T, ((0, 0), (0, EPAD - E)))

    Wf0a, Wf0b, Wf0c = Wf0[:D], Wf0[D:2 * D], Wf0[2 * D:]
    Wn0a, Wn0b = Wn0[:D], Wn0[D:]
    r = lambda v: v.reshape(1, D)
    grid0 = grid_nfeat[:N_MESH]

    mtab, gtab = _prep(mesh_nfeat, grid0, Wf0a, Wf0b)
    zeros = jnp.zeros((N_MESH, D), F32)
    aggs = []
    for s in range(NSLICE):
        gath_m, gath_g = _sc_gather(src4[s], dst4[s], mtab, gtab)
        e_upd = _edge(s, eft_p, gath_m, gath_g, We0, r(be0), We1, r(be1),
                      r(ge), r(bge), Wf0c, r(bf0), Wf1, r(bf1),
                      r(gf), r(bgf))
        aggp = _sc_scatter(e_upd, dst4[s], zeros)
        aggs.extend([aggp[0], aggp[1]])
    out_plain = _node_plain(grid_nfeat, Wn0a, r(bn0), Wn1, r(bn1),
                            r(gn), r(bgn))
    out_agg = _node_agg(grid0, aggs, Wn0a, Wn0b, r(bn0),
                        Wn1, r(bn1), r(gn), r(bgn))
    return lax.dynamic_update_slice(out_plain, out_agg, (0, 0))


# speedup vs baseline: 1.2472x; 1.0417x over previous
import functools

import jax
import jax.numpy as jnp
from jax import lax
from jax.experimental import pallas as pl
from jax.experimental.pallas import tpu as pltpu
from jax.experimental.pallas import tpu_sc as plsc

F32 = jnp.float32

E = 600000
N_GRID = 100000
N_MESH = 10000
D = 128
DE = 4

NC = 2
NS = 16
NW = NC * NS

CHUNK = 128
NCHUNK = 147
PER_TILE = CHUNK * NCHUNK
EPAD = NW * PER_TILE

NSLICE = 7
NCS = NCHUNK // NSLICE
ES = EPAD // NSLICE

B_EDGE = 2048
B_NODE = 1000


def _ln(h, g, b):
    mu = jnp.mean(h, axis=-1, keepdims=True)
    var = jnp.mean((h - mu) ** 2, axis=-1, keepdims=True)
    return g * (h - mu) / jnp.sqrt(var + 1e-5) + b


def _dot(a, b):
    return jnp.dot(a, b, preferred_element_type=F32)


def _prep_body(mesh_ref, grid0_ref, wa_ref, wb_ref, mp_ref, gp_ref):
    mp_ref[...] = _dot(mesh_ref[...], wa_ref[...])
    gp_ref[...] = _dot(grid0_ref[...], wb_ref[...])


def _prep(mesh_nfeat, grid0, Wf0a, Wf0b):
    nblk = N_MESH // B_NODE
    return pl.pallas_call(
        _prep_body,
        grid=(nblk,),
        in_specs=[
            pl.BlockSpec((B_NODE, D), lambda i: (i, 0)),
            pl.BlockSpec((B_NODE, D), lambda i: (i, 0)),
            pl.BlockSpec((D, D), lambda i: (0, 0)),
            pl.BlockSpec((D, D), lambda i: (0, 0)),
        ],
        out_specs=[
            pl.BlockSpec((B_NODE, D), lambda i: (i, 0)),
            pl.BlockSpec((B_NODE, D), lambda i: (i, 0)),
        ],
        out_shape=[
            jax.ShapeDtypeStruct((N_MESH, D), F32),
            jax.ShapeDtypeStruct((N_MESH, D), F32),
        ],
    )(mesh_nfeat, grid0, Wf0a, Wf0b)


def _sc_gather_body(src3_hbm, dst3_hbm, mtab_hbm, gtab_hbm,
                    outm_hbm, outg_hbm,
                    idx_s, idx_d, bm0, bm1, bg0, bg1,
                    sm0, sm1, sg0, sg1, swm0, swm1, swg0, swg1):
    cid = lax.axis_index("c")
    sid = lax.axis_index("s")
    wid = sid * NC + cid
    base = wid * (NCS * CHUNK)

    pltpu.sync_copy(src3_hbm.at[wid], idx_s)
    pltpu.sync_copy(dst3_hbm.at[wid], idx_d)

    bm = (bm0, bm1)
    bg = (bg0, bg1)
    sm = (sm0, sm1)
    sg = (sg0, sg1)
    swm = (swm0, swm1)
    swg = (swg0, swg1)
    h = {}

    h["m", 0] = pltpu.async_copy(mtab_hbm.at[idx_s.at[0]], bm[0], sm[0])
    h["g", 0] = pltpu.async_copy(gtab_hbm.at[idx_d.at[0]], bg[0], sg[0])
    for g in range(NCS):
        sl = g % 2
        nsl = (g + 1) % 2
        if g + 1 < NCS:
            if g >= 1:
                h["wm", g - 1].wait()
                h["wg", g - 1].wait()
            h["m", g + 1] = pltpu.async_copy(
                mtab_hbm.at[idx_s.at[g + 1]], bm[nsl], sm[nsl])
            h["g", g + 1] = pltpu.async_copy(
                gtab_hbm.at[idx_d.at[g + 1]], bg[nsl], sg[nsl])
        h["m", g].wait()
        h["g", g].wait()
        off = base + g * CHUNK
        h["wm", g] = pltpu.async_copy(
            bm[sl], outm_hbm.at[pl.ds(off, CHUNK)], swm[sl])
        h["wg", g] = pltpu.async_copy(
            bg[sl], outg_hbm.at[pl.ds(off, CHUNK)], swg[sl])
    h["wm", NCS - 2].wait()
    h["wg", NCS - 2].wait()
    h["wm", NCS - 1].wait()
    h["wg", NCS - 1].wait()


def _sc_gather(src3, dst3, mtab, gtab):
    mesh = plsc.VectorSubcoreMesh(
        core_axis_name="c", subcore_axis_name="s", num_cores=NC,
        num_subcores=NS)
    f = pl.kernel(
        _sc_gather_body,
        out_type=[
            jax.ShapeDtypeStruct((ES, D), F32),
            jax.ShapeDtypeStruct((ES, D), F32),
        ],
        mesh=mesh,
        compiler_params=pltpu.CompilerParams(use_tc_tiling_on_sc=True),
        scratch_types=[
            pltpu.VMEM((NCS, CHUNK), jnp.int32),
            pltpu.VMEM((NCS, CHUNK), jnp.int32),
            pltpu.VMEM((CHUNK, D), F32),
            pltpu.VMEM((CHUNK, D), F32),
            pltpu.VMEM((CHUNK, D), F32),
            pltpu.VMEM((CHUNK, D), F32),
            pltpu.SemaphoreType.DMA,
            pltpu.SemaphoreType.DMA,
            pltpu.SemaphoreType.DMA,
            pltpu.SemaphoreType.DMA,
            pltpu.SemaphoreType.DMA,
            pltpu.SemaphoreType.DMA,
            pltpu.SemaphoreType.DMA,
            pltpu.SemaphoreType.DMA,
        ],
    )
    return f(src3, dst3, mtab, gtab)


def _edge_body(base_rows, eft_ref, gm_ref, gg_ref,
               We0_ref, be0_ref, We1_ref, be1_ref, ge_ref, bge_ref,
               Wf0c_ref, bf0_ref, Wf1_ref, bf1_ref, gf_ref, bgf_ref,
               out_ref):
    i = pl.program_id(0)
    emb = lax.dot_general(eft_ref[...], We0_ref[...],
                          (((0,), (0,)), ((), ())),
                          preferred_element_type=F32)
    u = jax.nn.silu(emb + be0_ref[...])
    h = _dot(u, We1_ref[...]) + be1_ref[...]
    efeat = _ln(h, ge_ref[...], bge_ref[...])
    pre = (_dot(efeat, Wf0c_ref[...]) + bf0_ref[...]
           + gm_ref[...] + gg_ref[...])
    h2 = _dot(jax.nn.silu(pre), Wf1_ref[...]) + bf1_ref[...]
    e_upd = _ln(h2, gf_ref[...], bgf_ref[...])
    rows = (base_rows + i * B_EDGE
            + lax.broadcasted_iota(jnp.int32, (B_EDGE, 1), 0))
    out_ref[...] = jnp.where(rows < E, e_upd, 0.0)


def _edge(s, eft_p, gath_m, gath_g, We0, be0, We1, be1, ge, bge, Wf0c, bf0,
          Wf1, bf1, gf, bgf):
    nblk = ES // B_EDGE
    base_blk = s * nblk
    full = lambda shape: pl.BlockSpec(shape, lambda i: (0, 0))
    return pl.pallas_call(
        functools.partial(_edge_body, s * ES),
        grid=(nblk,),
        in_specs=[
            pl.BlockSpec((DE, B_EDGE), lambda i: (0, base_blk + i)),
            pl.BlockSpec((B_EDGE, D), lambda i: (i, 0)),
            pl.BlockSpec((B_EDGE, D), lambda i: (i, 0)),
            full((DE, D)), full((1, D)), full((D, D)), full((1, D)),
            full((1, D)), full((1, D)),
            full((D, D)), full((1, D)), full((D, D)), full((1, D)),
            full((1, D)), full((1, D)),
        ],
        out_specs=pl.BlockSpec((B_EDGE, D), lambda i: (i, 0)),
        out_shape=jax.ShapeDtypeStruct((ES, D), F32),
    )(eft_p, gath_m, gath_g, We0, be0, We1, be1, ge, bge, Wf0c, bf0,
      Wf1, bf1, gf, bgf)


def _sc_scatter_body(eupd_hbm, dst3_hbm, zeros_hbm, agg_hbm,
                     idx_t, be0, be1, agg_s, se0, se1):
    cid = lax.axis_index("c")
    sid = lax.axis_index("s")
    wid = sid * NC + cid
    base = wid * (NCS * CHUNK)

    pltpu.sync_copy(dst3_hbm.at[wid], idx_t)

    @pl.when(sid == 0)
    def _():
        pltpu.sync_copy(zeros_hbm, agg_s)

    plsc.subcore_barrier()

    be = (be0, be1)
    se = (se0, se1)
    h = {}
    h[0] = pltpu.async_copy(eupd_hbm.at[pl.ds(base, CHUNK)], be[0], se[0])
    for g in range(NCS):
        sl = g % 2
        nsl = (g + 1) % 2
        if g + 1 < NCS:
            h[g + 1] = pltpu.async_copy(
                eupd_hbm.at[pl.ds(base + (g + 1) * CHUNK, CHUNK)],
                be[nsl], se[nsl])
        h[g].wait()
        pltpu.sync_copy(be[sl], agg_s.at[idx_t.at[g]], add=True)

    plsc.subcore_barrier()

    @pl.when(sid == 0)
    def _():
        pltpu.sync_copy(agg_s, agg_hbm.at[cid])


def _sc_scatter(e_upd, dst3, zeros):
    mesh = plsc.VectorSubcoreMesh(
        core_axis_name="c", subcore_axis_name="s", num_cores=NC,
        num_subcores=NS)
    f = pl.kernel(
        _sc_scatter_body,
        out_type=jax.ShapeDtypeStruct((NC, N_MESH, D), F32),
        mesh=mesh,
        compiler_params=pltpu.CompilerParams(use_tc_tiling_on_sc=True),
        scratch_types=[
            pltpu.VMEM((NCS, CHUNK), jnp.int32),
            pltpu.VMEM((CHUNK, D), F32),
            pltpu.VMEM((CHUNK, D), F32),
            pltpu.VMEM_SHARED((N_MESH, D), F32),
            pltpu.SemaphoreType.DMA,
            pltpu.SemaphoreType.DMA,
        ],
    )
    return f(e_upd, dst3, zeros)


NAGG = NSLICE * NC


def _node_plain_body(grid_ref, Wn0a_ref, bn0_ref, Wn1_ref, bn1_ref,
                     gn_ref, bgn_ref, out_ref):
    g = grid_ref[...]
    pre = _dot(g, Wn0a_ref[...]) + bn0_ref[...]
    h = _dot(jax.nn.silu(pre), Wn1_ref[...]) + bn1_ref[...]
    out_ref[...] = g + _ln(h, gn_ref[...], bgn_ref[...])


def _node_plain(grid_nfeat, Wn0a, bn0, Wn1, bn1, gn, bgn):
    nblk = N_GRID // B_NODE
    full = lambda shape: pl.BlockSpec(shape, lambda i: (0, 0))
    return pl.pallas_call(
        _node_plain_body,
        grid=(nblk,),
        in_specs=[
            pl.BlockSpec((B_NODE, D), lambda i: (i, 0)),
            full((D, D)), full((1, D)), full((D, D)),
            full((1, D)), full((1, D)), full((1, D)),
        ],
        out_specs=pl.BlockSpec((B_NODE, D), lambda i: (i, 0)),
        out_shape=jax.ShapeDtypeStruct((N_GRID, D), F32),
    )(grid_nfeat, Wn0a, bn0, Wn1, bn1, gn, bgn)


def _node_agg_body(grid_ref, *rest):
    agg_refs = rest[:NAGG]
    (Wn0a_ref, Wn0b_ref, bn0_ref, Wn1_ref, bn1_ref,
     gn_ref, bgn_ref, out_ref) = rest[NAGG:]
    g = grid_ref[...]
    agg = agg_refs[0][...]
    for a in agg_refs[1:]:
        agg = agg + a[...]
    pre = _dot(g, Wn0a_ref[...]) + _dot(agg, Wn0b_ref[...]) + bn0_ref[...]
    h = _dot(jax.nn.silu(pre), Wn1_ref[...]) + bn1_ref[...]
    out_ref[...] = g + _ln(h, gn_ref[...], bgn_ref[...])


def _node_agg(grid0, aggs, Wn0a, Wn0b, bn0, Wn1, bn1, gn, bgn):
    nblk = N_MESH // B_NODE
    full = lambda shape: pl.BlockSpec(shape, lambda i: (0, 0))
    blk = pl.BlockSpec((B_NODE, D), lambda i: (i, 0))
    return pl.pallas_call(
        _node_agg_body,
        grid=(nblk,),
        in_specs=[
            blk,
            *([blk] * NAGG),
            full((D, D)), full((D, D)), full((1, D)), full((D, D)),
            full((1, D)), full((1, D)), full((1, D)),
        ],
        out_specs=blk,
        out_shape=jax.ShapeDtypeStruct((N_MESH, D), F32),
    )(grid0, *aggs, Wn0a, Wn0b, bn0, Wn1, bn1, gn, bgn)


def kernel(grid_nfeat, mesh_nfeat, edge_index, mesh2grid_efeat,
           We0, be0, We1, be1, ge, bge,
           Wf0, bf0, Wf1, bf1, gf, bgf,
           Wn0, bn0, Wn1, bn1, gn, bgn):
    src = edge_index[0].astype(jnp.int32)
    dst = edge_index[1].astype(jnp.int32)
    src4 = jnp.pad(src, (0, EPAD - E)).reshape(NSLICE, NW, NCS, CHUNK)
    dst4 = jnp.pad(dst, (0, EPAD - E)).reshape(NSLICE, NW, NCS, CHUNK)
    eft_p = jnp.pad(mesh2grid_efeat.T, ((0, 0), (0, EPAD - E)))

    Wf0a, Wf0b, Wf0c = Wf0[:D], Wf0[D:2 * D], Wf0[2 * D:]
    Wn0a, Wn0b = Wn0[:D], Wn0[D:]
    r = lambda v: v.reshape(1, D)
    grid0 = grid_nfeat[:N_MESH]

    mtab, gtab = _prep(mesh_nfeat, grid0, Wf0a, Wf0b)
    zeros = jnp.zeros((N_MESH, D), F32)
    aggs = []
    for s in range(NSLICE):
        gath_m, gath_g = _sc_gather(src4[s], dst4[s], mtab, gtab)
        e_upd = _edge(s, eft_p, gath_m, gath_g, We0, r(be0), We1, r(be1),
                      r(ge), r(bge), Wf0c, r(bf0), Wf1, r(bf1),
                      r(gf), r(bgf))
        aggp = _sc_scatter(e_upd, dst4[s], zeros)
        aggs.extend([aggp[0], aggp[1]])
    out_plain = _node_plain(grid_nfeat, Wn0a, r(bn0), Wn1, r(bn1),
                            r(gn), r(bgn))
    out_agg = _node_agg(grid0, aggs, Wn0a, Wn0b, r(bn0),
                        Wn1, r(bn1), r(gn), r(bgn))
    return lax.dynamic_update_slice(out_plain, out_agg, (0, 0))
